# Initial kernel scaffold; baseline (speedup 1.0000x reference)
import jax
import jax.numpy as jnp
from jax.experimental import pallas as pl


def kernel(entity_emb, edge_index, edge_type, weight):
    raise NotImplementedError("write your pallas kernel here")


# speedup vs baseline: 3.9041x; 3.9041x over previous
import functools

import jax
import jax.numpy as jnp
from jax import lax
from jax.experimental import pallas as pl
from jax.experimental.pallas import tpu as pltpu
from jax.experimental.pallas import tpu_sc as plsc

D = 128
NREL = 32
NC, NS = 2, 16
NW = NC * NS
CHUNK = 128
ROWS_PER_TILE = 640
N_PAD = NS * ROWS_PER_TILE


def _sc_partials(entity_emb, head, tail, etype, weight):
    n_edges = head.shape[0]
    n_chunks = n_edges // CHUNK

    mesh = plsc.VectorSubcoreMesh(core_axis_name="c", subcore_axis_name="s")

    @functools.partial(
        pl.kernel,
        out_type=(
            jax.ShapeDtypeStruct((NC, N_PAD, D), jnp.float32),
            jax.ShapeDtypeStruct((NC, N_PAD), jnp.float32),
        ),
        mesh=mesh,
        scratch_types=[
            pltpu.VMEM_SHARED((N_PAD, D), jnp.float32),
            pltpu.VMEM_SHARED((N_PAD,), jnp.float32),
            pltpu.VMEM((NREL, D), jnp.float32),
            pltpu.VMEM((CHUNK,), jnp.int32),
            pltpu.VMEM((CHUNK,), jnp.int32),
            pltpu.VMEM((CHUNK,), jnp.int32),
            pltpu.VMEM((CHUNK, D), jnp.float32),
            pltpu.VMEM((CHUNK,), jnp.float32),
            pltpu.VMEM((ROWS_PER_TILE,), jnp.float32),
            pltpu.SemaphoreType.DMA,
        ],
    )
    def agg(emb_h, head_h, tail_h, type_h, w_h, psum_h, pcnt_h,
            acc_s, cnt_s, w_v, tail_v, head_v, type_v, rows_v, ones_v,
            zcnt_v, sem):
        cid = lax.axis_index("c")
        sid = lax.axis_index("s")
        wid = sid * NC + cid

        def zrow(i, carry):
            for k in range(D // 16):
                rows_v[i, pl.ds(k * 16, 16)] = jnp.zeros((16,), jnp.float32)
            return carry
        lax.fori_loop(0, CHUNK, zrow, 0)

        def zsmall(i, carry):
            ones_v[pl.ds(i * 16, 16)] = jnp.ones((16,), jnp.float32)
            return carry
        lax.fori_loop(0, CHUNK // 16, zsmall, 0)

        def zcnt(i, carry):
            zcnt_v[pl.ds(i * 16, 16)] = jnp.zeros((16,), jnp.float32)
            return carry
        lax.fori_loop(0, ROWS_PER_TILE // 16, zcnt, 0)

        pltpu.sync_copy(w_h, w_v)

        base_row = sid * ROWS_PER_TILE
        for b in range(ROWS_PER_TILE // CHUNK):
            pltpu.sync_copy(rows_v, acc_s.at[pl.ds(base_row + b * CHUNK, CHUNK)])
        pltpu.sync_copy(zcnt_v, cnt_s.at[pl.ds(base_row, ROWS_PER_TILE)])
        plsc.subcore_barrier()

        my_chunks = n_chunks // NW + jnp.where(wid < (n_chunks % NW), 1, 0)

        def chunk_body(j, carry):
            chunk_id = wid + j * NW
            base = chunk_id * CHUNK
            pltpu.sync_copy(tail_h.at[pl.ds(base, CHUNK)], tail_v)
            pltpu.sync_copy(head_h.at[pl.ds(base, CHUNK)], head_v)
            pltpu.sync_copy(type_h.at[pl.ds(base, CHUNK)], type_v)
            pltpu.async_copy(emb_h.at[tail_v], rows_v, sem).wait()

            def group_body(g, c2):
                tvec = type_v[pl.ds(g * 16, 16)]
                for l in range(16):
                    t = tvec[l]
                    e = g * 16 + l
                    for k in range(D // 16):
                        sl = pl.ds(k * 16, 16)
                        rows_v[e, sl] = rows_v[e, sl] * w_v[t, sl]
                return c2
            lax.fori_loop(0, CHUNK // 16, group_body, 0)

            pltpu.sync_copy(rows_v, acc_s.at[head_v], add=True)
            pltpu.sync_copy(ones_v, cnt_s.at[head_v], add=True)
            return carry
        lax.fori_loop(0, my_chunks, chunk_body, 0)

        plsc.subcore_barrier()

        pltpu.sync_copy(acc_s.at[pl.ds(base_row, ROWS_PER_TILE)],
                        psum_h.at[cid, pl.ds(base_row, ROWS_PER_TILE)])
        pltpu.sync_copy(cnt_s.at[pl.ds(base_row, ROWS_PER_TILE)],
                        pcnt_h.at[cid, pl.ds(base_row, ROWS_PER_TILE)])

    return agg(entity_emb, head, tail, etype, weight)


def _combine_kernel(p_ref, c_ref, o_ref):
    s = p_ref[0] + p_ref[1]
    c = c_ref[0] + c_ref[1]
    c = jnp.maximum(c, 1.0)
    o_ref[...] = s / c


def _tc_combine(psum, pcnt):
    blocks = N_PAD // D
    pcnt2 = pcnt.reshape(NC, N_PAD, 1)
    out = pl.pallas_call(
        _combine_kernel,
        grid=(blocks,),
        in_specs=[
            pl.BlockSpec((NC, D, D), lambda i: (0, i, 0)),
            pl.BlockSpec((NC, D, 1), lambda i: (0, i, 0)),
        ],
        out_specs=pl.BlockSpec((D, D), lambda i: (i, 0)),
        out_shape=jax.ShapeDtypeStruct((N_PAD, D), jnp.float32),
    )(psum, pcnt2)
    return out


def kernel(entity_emb, edge_index, edge_type, weight):
    n_entities = entity_emb.shape[0]
    head = edge_index[0].astype(jnp.int32)
    tail = edge_index[1].astype(jnp.int32)
    etype = edge_type.astype(jnp.int32)
    psum, pcnt = _sc_partials(entity_emb, head, tail, etype, weight)
    out = _tc_combine(psum, pcnt)
    return out[:n_entities]
